# Initial kernel scaffold; baseline (speedup 1.0000x reference)
import jax
import jax.numpy as jnp
from jax.experimental import pallas as pl


def kernel(x, edge_index, edge_weight, batch, W1, b1, W2, b2):
    raise NotImplementedError("write your pallas kernel here")


# speedup vs baseline: 31.7781x; 31.7781x over previous
import jax
import jax.numpy as jnp
from jax import lax
from jax.experimental import pallas as pl
from jax.experimental.pallas import tpu as pltpu
from jax.experimental.pallas import tpu_sc as plsc

N_NODES = 10000
N_PAD = 10240
HID = 16
NC = 2
NS = 16
NW = NC * NS
LW = 128
CROWS = 16
ZROWS = N_PAD // NS

_MESH = plsc.VectorSubcoreMesh(core_axis_name="c", subcore_axis_name="s")
_SC_PARAMS = pltpu.CompilerParams(use_tc_tiling_on_sc=False)


def _deg_call(col, ew):
    rows = col.shape[0]
    rpt = rows // NW

    def body(col_hbm, ew_hbm, out_hbm, colv, ewv, zb, degsp):
        c = lax.axis_index("c")
        s = lax.axis_index("s")
        wid = c * NS + s

        @pl.loop(0, ZROWS, step=16)
        def _zero(i):
            zb[pl.ds(i, 16)] = jnp.zeros((16,), jnp.float32)

        pltpu.sync_copy(zb, degsp.at[pl.ds(s * ZROWS, ZROWS)])
        plsc.subcore_barrier()

        base = wid * rpt

        @pl.loop(0, rpt, step=CROWS)
        def _chunk(r0):
            pltpu.sync_copy(col_hbm.at[pl.ds(base + r0, CROWS)], colv)
            pltpu.sync_copy(ew_hbm.at[pl.ds(base + r0, CROWS)], ewv)
            for j in range(CROWS):
                pltpu.sync_copy(ewv.at[j], degsp.at[colv.at[j]], add=True)

        plsc.subcore_barrier()
        pltpu.sync_copy(degsp.at[pl.ds(s * ZROWS, ZROWS)],
                        out_hbm.at[c, pl.ds(s * ZROWS, ZROWS)])

    return pl.kernel(
        body,
        out_type=jax.ShapeDtypeStruct((NC, N_PAD), jnp.float32),
        mesh=_MESH,
        compiler_params=_SC_PARAMS,
        scratch_types=[
            pltpu.VMEM((CROWS, LW), jnp.int32),
            pltpu.VMEM((CROWS, LW), jnp.float32),
            pltpu.VMEM((ZROWS,), jnp.float32),
            pltpu.VMEM_SHARED((N_PAD,), jnp.float32),
        ],
    )(col, ew)


def _edge_call(y, row, col, ew):
    rows = col.shape[0]
    rpt = rows // NW

    def body(y_hbm, row_hbm, col_hbm, ew_hbm, out_hbm,
             rowv, colv, ewv, msg, zb, accsp):
        c = lax.axis_index("c")
        s = lax.axis_index("s")
        wid = c * NS + s

        @pl.loop(0, ZROWS)
        def _zero(i):
            zb[i, :] = jnp.zeros((HID,), jnp.float32)

        pltpu.sync_copy(zb, accsp.at[pl.ds(s * ZROWS, ZROWS)])
        plsc.subcore_barrier()

        base = wid * rpt

        @pl.loop(0, rpt, step=CROWS)
        def _chunk(r0):
            pltpu.sync_copy(row_hbm.at[pl.ds(base + r0, CROWS)], rowv)
            pltpu.sync_copy(col_hbm.at[pl.ds(base + r0, CROWS)], colv)
            pltpu.sync_copy(ew_hbm.at[pl.ds(base + r0, CROWS)], ewv)
            for j in range(CROWS):
                pltpu.sync_copy(y_hbm.at[rowv.at[j]],
                                msg.at[pl.ds(j * LW, LW)])

            @pl.loop(0, CROWS)
            def _sj(j):
                @pl.loop(0, LW, step=16)
                def _si(i0):
                    ew16 = ewv[j, pl.ds(i0, 16)]
                    e0 = j * LW + i0
                    for k in range(16):
                        msg[e0 + k] = msg[e0 + k] * ew16[k]

            for j in range(CROWS):
                pltpu.sync_copy(msg.at[pl.ds(j * LW, LW)],
                                accsp.at[colv.at[j]], add=True)

        plsc.subcore_barrier()
        pltpu.sync_copy(accsp.at[pl.ds(s * ZROWS, ZROWS)],
                        out_hbm.at[c, pl.ds(s * ZROWS, ZROWS)])

    return pl.kernel(
        body,
        out_type=jax.ShapeDtypeStruct((NC, N_PAD, HID), jnp.float32),
        mesh=_MESH,
        compiler_params=_SC_PARAMS,
        scratch_types=[
            pltpu.VMEM((CROWS, LW), jnp.int32),
            pltpu.VMEM((CROWS, LW), jnp.int32),
            pltpu.VMEM((CROWS, LW), jnp.float32),
            pltpu.VMEM((CROWS * LW, HID), jnp.float32),
            pltpu.VMEM((ZROWS, HID), jnp.float32),
            pltpu.VMEM_SHARED((N_PAD, HID), jnp.float32),
        ],
    )(y, row, col, ew)


def _dinv(degt):
    deg = jnp.sum(degt, axis=1, keepdims=True) + 1.0
    return jnp.where(deg > 0, lax.rsqrt(deg), 0.0)


def _mm_body(x_ref, w_ref, o_ref):
    o_ref[...] = jnp.dot(x_ref[...], w_ref[...],
                         preferred_element_type=jnp.float32)


def _yscale_body(degt_ref, xw_ref, o_ref):
    o_ref[...] = _dinv(degt_ref[...]) * xw_ref[...]


def _comb1_body(acc_ref, y_ref, degt_ref, b_ref, w_ref, o_ref):
    dinv = _dinv(degt_ref[...])
    tot = acc_ref[0] + acc_ref[1] + y_ref[...]
    h = jnp.maximum(dinv * tot + b_ref[...], 0.0)
    o_ref[...] = dinv * jnp.dot(h, w_ref[...],
                                preferred_element_type=jnp.float32)


def _comb2_body(acc_ref, y_ref, degt_ref, b_ref, o_ref):
    dinv = _dinv(degt_ref[...])
    tot = acc_ref[0] + acc_ref[1] + y_ref[...]
    o_ref[...] = jnp.maximum(dinv * tot + b_ref[...], 0.0)


def _tc(body, out_shape, *args):
    return pl.pallas_call(
        body, out_shape=jax.ShapeDtypeStruct(out_shape, jnp.float32))(*args)


def kernel(x, edge_index, edge_weight, batch, W1, b1, W2, b2):
    del batch
    x = x.astype(jnp.float32)
    e = edge_weight.shape[0]
    ep = -(-e // (NW * CROWS * LW)) * (NW * CROWS * LW)
    npadex = ep - e
    pad_nodes = (jnp.arange(npadex, dtype=jnp.int32) % (N_PAD - N_NODES)
                 + N_NODES)
    row = jnp.concatenate([edge_index[0], pad_nodes]).reshape(ep // LW, LW)
    col = jnp.concatenate([edge_index[1], pad_nodes]).reshape(ep // LW, LW)
    ew = jnp.concatenate(
        [edge_weight.astype(jnp.float32), jnp.zeros((npadex,), jnp.float32)]
    ).reshape(ep // LW, LW)
    xpad = jnp.pad(x, ((0, N_PAD - N_NODES), (0, 0)))

    degp = _deg_call(col, ew)
    xw1 = _tc(_mm_body, (N_PAD, HID), xpad, W1)
    degt = degp.T
    y1 = _tc(_yscale_body, (N_PAD, HID), degt, xw1)
    acc1 = _edge_call(y1, row, col, ew)
    y2 = _tc(_comb1_body, (N_PAD, HID), acc1, y1, degt,
             b1.reshape(1, HID).astype(jnp.float32), W2)
    acc2 = _edge_call(y2, row, col, ew)
    out = _tc(_comb2_body, (N_PAD, HID), acc2, y2, degt,
              b2.reshape(1, HID).astype(jnp.float32))
    return out[:N_NODES]


# speedup vs baseline: 53.4644x; 1.6824x over previous
import jax
import jax.numpy as jnp
from jax import lax
from jax.experimental import pallas as pl
from jax.experimental.pallas import tpu as pltpu
from jax.experimental.pallas import tpu_sc as plsc

N_NODES = 10000
N_PAD = 10240
HID = 16
NC = 2
NS = 16
NW = NC * NS
LW = 128
CROWS = 16
ZROWS = N_PAD // NS

_MESH = plsc.VectorSubcoreMesh(core_axis_name="c", subcore_axis_name="s")
_SC_PARAMS = pltpu.CompilerParams(use_tc_tiling_on_sc=False)


def _deg_call(col, ew):
    rows = col.shape[0]
    rpt = rows // NW

    def body(col_hbm, ew_hbm, out_hbm, colv, ewv, zb, degsp):
        c = lax.axis_index("c")
        s = lax.axis_index("s")
        wid = c * NS + s

        @pl.loop(0, ZROWS, step=16)
        def _zero(i):
            zb[pl.ds(i, 16)] = jnp.zeros((16,), jnp.float32)

        pltpu.sync_copy(zb, degsp.at[pl.ds(s * ZROWS, ZROWS)])
        plsc.subcore_barrier()

        base = wid * rpt

        @pl.loop(0, rpt, step=CROWS)
        def _chunk(r0):
            pltpu.sync_copy(col_hbm.at[pl.ds(base + r0, CROWS)], colv)
            pltpu.sync_copy(ew_hbm.at[pl.ds(base + r0, CROWS)], ewv)
            for j in range(CROWS):
                pltpu.sync_copy(ewv.at[j], degsp.at[colv.at[j]], add=True)

        plsc.subcore_barrier()
        pltpu.sync_copy(degsp.at[pl.ds(s * ZROWS, ZROWS)],
                        out_hbm.at[c, pl.ds(s * ZROWS, ZROWS)])

    return pl.kernel(
        body,
        out_type=jax.ShapeDtypeStruct((NC, N_PAD), jnp.float32),
        mesh=_MESH,
        compiler_params=_SC_PARAMS,
        scratch_types=[
            pltpu.VMEM((CROWS, LW), jnp.int32),
            pltpu.VMEM((CROWS, LW), jnp.float32),
            pltpu.VMEM((ZROWS,), jnp.float32),
            pltpu.VMEM_SHARED((N_PAD,), jnp.float32),
        ],
    )(col, ew)


def _edge_call(y, row, col, ew):
    rows = col.shape[0]
    rpt = rows // NW
    nchunk = rpt // CROWS

    def body(y_hbm, row_hbm, col_hbm, ew_hbm, out_hbm,
             rowv0, rowv1, colv0, colv1, ewv0, ewv1, msg0, msg1,
             zb, accsp, gsem0, gsem1, ssem0, ssem1):
        rowv = [rowv0, rowv1]
        colv = [colv0, colv1]
        ewv = [ewv0, ewv1]
        msg = [msg0, msg1]
        gsem = [gsem0, gsem1]
        ssem = [ssem0, ssem1]
        c = lax.axis_index("c")
        s = lax.axis_index("s")
        wid = c * NS + s

        @pl.loop(0, ZROWS)
        def _zero(i):
            zb[i, :] = jnp.zeros((HID,), jnp.float32)

        pltpu.sync_copy(zb, accsp.at[pl.ds(s * ZROWS, ZROWS)])
        plsc.subcore_barrier()

        base = wid * rpt

        def issue_idx(k):
            p = k & 1
            pltpu.sync_copy(row_hbm.at[pl.ds(base + k * CROWS, CROWS)],
                            rowv[p])
            pltpu.sync_copy(col_hbm.at[pl.ds(base + k * CROWS, CROWS)],
                            colv[p])
            pltpu.sync_copy(ew_hbm.at[pl.ds(base + k * CROWS, CROWS)],
                            ewv[p])

        def issue_gathers(k):
            p = k & 1
            return [pltpu.async_copy(y_hbm.at[rowv[p].at[j]],
                                     msg[p].at[pl.ds(j * LW, LW)], gsem[p])
                    for j in range(CROWS)]

        def issue_scatters(k):
            p = k & 1
            return [pltpu.async_copy(msg[p].at[pl.ds(j * LW, LW)],
                                     accsp.at[colv[p].at[j]], ssem[p],
                                     add=True)
                    for j in range(CROWS)]

        def scale(k):
            p = k & 1

            @pl.loop(0, CROWS)
            def _sj(j):
                @pl.loop(0, LW, step=16)
                def _si(i0):
                    ew16 = ewv[p][j, pl.ds(i0, 16)]
                    e0 = j * LW + i0
                    for t in range(16):
                        msg[p][e0 + t] = msg[p][e0 + t] * ew16[t]

        issue_idx(0)
        gd = {0: issue_gathers(0)}
        sd = {}
        for k in range(nchunk):
            if k + 1 < nchunk:
                if k - 1 in sd:
                    for d in sd.pop(k - 1):
                        d.wait()
                issue_idx(k + 1)
                gd[k + 1] = issue_gathers(k + 1)
            for d in gd.pop(k):
                d.wait()
            scale(k)
            sd[k] = issue_scatters(k)
        for kk in sorted(sd):
            for d in sd[kk]:
                d.wait()

        plsc.subcore_barrier()
        pltpu.sync_copy(accsp.at[pl.ds(s * ZROWS, ZROWS)],
                        out_hbm.at[c, pl.ds(s * ZROWS, ZROWS)])

    return pl.kernel(
        body,
        out_type=jax.ShapeDtypeStruct((NC, N_PAD, HID), jnp.float32),
        mesh=_MESH,
        compiler_params=_SC_PARAMS,
        scratch_types=[
            pltpu.VMEM((CROWS, LW), jnp.int32),
            pltpu.VMEM((CROWS, LW), jnp.int32),
            pltpu.VMEM((CROWS, LW), jnp.int32),
            pltpu.VMEM((CROWS, LW), jnp.int32),
            pltpu.VMEM((CROWS, LW), jnp.float32),
            pltpu.VMEM((CROWS, LW), jnp.float32),
            pltpu.VMEM((CROWS * LW, HID), jnp.float32),
            pltpu.VMEM((CROWS * LW, HID), jnp.float32),
            pltpu.VMEM((ZROWS, HID), jnp.float32),
            pltpu.VMEM_SHARED((N_PAD, HID), jnp.float32),
            pltpu.SemaphoreType.DMA,
            pltpu.SemaphoreType.DMA,
            pltpu.SemaphoreType.DMA,
            pltpu.SemaphoreType.DMA,
        ],
    )(y, row, col, ew)


def _dinv(degt):
    deg = jnp.sum(degt, axis=1, keepdims=True) + 1.0
    return jnp.where(deg > 0, lax.rsqrt(deg), 0.0)


def _y1_body(x_ref, w_ref, degt_ref, o_ref):
    dinv = _dinv(degt_ref[...])
    xw = jnp.dot(x_ref[...], w_ref[...],
                 preferred_element_type=jnp.float32)
    o_ref[pl.ds(0, N_NODES)] = dinv[:N_NODES] * xw
    o_ref[pl.ds(N_NODES, N_PAD - N_NODES)] = jnp.zeros(
        (N_PAD - N_NODES, HID), jnp.float32)


def _comb1_body(acc_ref, y_ref, degt_ref, b_ref, w_ref, o_ref):
    dinv = _dinv(degt_ref[...])
    tot = acc_ref[0] + acc_ref[1] + y_ref[...]
    h = jnp.maximum(dinv * tot + b_ref[...], 0.0)
    o_ref[...] = dinv * jnp.dot(h, w_ref[...],
                                preferred_element_type=jnp.float32)


def _comb2_body(acc_ref, y_ref, degt_ref, b_ref, o_ref):
    dinv = _dinv(degt_ref[...])
    tot = acc_ref[0] + acc_ref[1] + y_ref[...]
    o_ref[...] = jnp.maximum(dinv * tot + b_ref[...], 0.0)


def _tc(body, out_shape, *args):
    return pl.pallas_call(
        body, out_shape=jax.ShapeDtypeStruct(out_shape, jnp.float32))(*args)


def kernel(x, edge_index, edge_weight, batch, W1, b1, W2, b2):
    del batch
    x = x.astype(jnp.float32)
    e = edge_weight.shape[0]
    ep = -(-e // (NW * CROWS * LW)) * (NW * CROWS * LW)
    npadex = ep - e
    pad_nodes = (jnp.arange(npadex, dtype=jnp.int32) % (N_PAD - N_NODES)
                 + N_NODES)
    row = jnp.concatenate([edge_index[0], pad_nodes]).reshape(ep // LW, LW)
    col = jnp.concatenate([edge_index[1], pad_nodes]).reshape(ep // LW, LW)
    ew = jnp.concatenate(
        [edge_weight.astype(jnp.float32), jnp.zeros((npadex,), jnp.float32)]
    ).reshape(ep // LW, LW)

    degp = _deg_call(col, ew)
    degt = degp.T
    y1 = _tc(_y1_body, (N_PAD, HID), x, W1, degt)
    acc1 = _edge_call(y1, row, col, ew)
    y2 = _tc(_comb1_body, (N_PAD, HID), acc1, y1, degt,
             b1.reshape(1, HID).astype(jnp.float32), W2)
    acc2 = _edge_call(y2, row, col, ew)
    out = _tc(_comb2_body, (N_PAD, HID), acc2, y2, degt,
              b2.reshape(1, HID).astype(jnp.float32))
    return out[:N_NODES]


# speedup vs baseline: 56.1540x; 1.0503x over previous
import jax
import jax.numpy as jnp
from jax import lax
from jax.experimental import pallas as pl
from jax.experimental.pallas import tpu as pltpu
from jax.experimental.pallas import tpu_sc as plsc

N_NODES = 10000
N_PAD = 10240
HID = 16
NC = 2
NS = 16
NW = NC * NS
LW = 128
CROWS = 16
ZROWS = N_PAD // NS

_MESH = plsc.VectorSubcoreMesh(core_axis_name="c", subcore_axis_name="s")
_SC_PARAMS = pltpu.CompilerParams(use_tc_tiling_on_sc=False)


def _deg_call(col, ew):
    ce = CROWS * LW
    rpt = col.shape[0] // NW

    def body(col_hbm, ew_hbm, out_hbm, colv, ewv, zb, degsp):
        c = lax.axis_index("c")
        s = lax.axis_index("s")
        wid = c * NS + s

        @pl.loop(0, ZROWS, step=16)
        def _zero(i):
            zb[pl.ds(i, 16)] = jnp.zeros((16,), jnp.float32)

        pltpu.sync_copy(zb, degsp.at[pl.ds(s * ZROWS, ZROWS)])
        plsc.subcore_barrier()

        base = wid * rpt

        @pl.loop(0, rpt, step=ce)
        def _chunk(e0):
            pltpu.sync_copy(col_hbm.at[pl.ds(base + e0, ce)], colv)
            pltpu.sync_copy(ew_hbm.at[pl.ds(base + e0, ce)], ewv)
            pltpu.sync_copy(ewv, degsp.at[colv], add=True)

        plsc.subcore_barrier()
        pltpu.sync_copy(degsp.at[pl.ds(s * ZROWS, ZROWS)],
                        out_hbm.at[c, pl.ds(s * ZROWS, ZROWS)])

    return pl.kernel(
        body,
        out_type=jax.ShapeDtypeStruct((NC, N_PAD), jnp.float32),
        mesh=_MESH,
        compiler_params=_SC_PARAMS,
        scratch_types=[
            pltpu.VMEM((CROWS * LW,), jnp.int32),
            pltpu.VMEM((CROWS * LW,), jnp.float32),
            pltpu.VMEM((ZROWS,), jnp.float32),
            pltpu.VMEM_SHARED((N_PAD,), jnp.float32),
        ],
    )(col, ew)


def _edge_call(y, row, col, ew):
    ce = CROWS * LW
    rpt = col.shape[0] // NW
    nchunk = rpt // ce

    def body(y_hbm, row_hbm, col_hbm, ew_hbm, out_hbm,
             rowv0, rowv1, colv0, colv1, ewv0, ewv1, msg0, msg1,
             zb, accsp, gsem0, gsem1, ssem0, ssem1):
        rowv = [rowv0, rowv1]
        colv = [colv0, colv1]
        ewv = [ewv0, ewv1]
        msg = [msg0, msg1]
        gsem = [gsem0, gsem1]
        ssem = [ssem0, ssem1]
        c = lax.axis_index("c")
        s = lax.axis_index("s")
        wid = c * NS + s

        @pl.loop(0, ZROWS)
        def _zero(i):
            zb[i, :] = jnp.zeros((HID,), jnp.float32)

        pltpu.sync_copy(zb, accsp.at[pl.ds(s * ZROWS, ZROWS)])
        plsc.subcore_barrier()

        base = wid * rpt

        def issue_idx(k):
            p = k & 1
            pltpu.sync_copy(row_hbm.at[pl.ds(base + k * ce, ce)], rowv[p])
            pltpu.sync_copy(col_hbm.at[pl.ds(base + k * ce, ce)], colv[p])
            pltpu.sync_copy(ew_hbm.at[pl.ds(base + k * ce, ce)], ewv[p])

        def issue_gathers(k):
            p = k & 1
            return [pltpu.async_copy(y_hbm.at[rowv[p]], msg[p], gsem[p])]

        def issue_scatters(k):
            p = k & 1
            return [pltpu.async_copy(msg[p], accsp.at[colv[p]], ssem[p],
                                     add=True)]

        def scale(k):
            p = k & 1

            @pl.loop(0, ce, step=16)
            def _si(i0):
                ew16 = ewv[p][pl.ds(i0, 16)]
                for t in range(16):
                    msg[p][i0 + t] = msg[p][i0 + t] * ew16[t]

        issue_idx(0)
        gd = {0: issue_gathers(0)}
        sd = {}
        for k in range(nchunk):
            if k + 1 < nchunk:
                if k - 1 in sd:
                    for d in sd.pop(k - 1):
                        d.wait()
                issue_idx(k + 1)
                gd[k + 1] = issue_gathers(k + 1)
            for d in gd.pop(k):
                d.wait()
            scale(k)
            sd[k] = issue_scatters(k)
        for kk in sorted(sd):
            for d in sd[kk]:
                d.wait()

        plsc.subcore_barrier()
        pltpu.sync_copy(accsp.at[pl.ds(s * ZROWS, ZROWS)],
                        out_hbm.at[c, pl.ds(s * ZROWS, ZROWS)])

    return pl.kernel(
        body,
        out_type=jax.ShapeDtypeStruct((NC, N_PAD, HID), jnp.float32),
        mesh=_MESH,
        compiler_params=_SC_PARAMS,
        scratch_types=[
            pltpu.VMEM((CROWS * LW,), jnp.int32),
            pltpu.VMEM((CROWS * LW,), jnp.int32),
            pltpu.VMEM((CROWS * LW,), jnp.int32),
            pltpu.VMEM((CROWS * LW,), jnp.int32),
            pltpu.VMEM((CROWS * LW,), jnp.float32),
            pltpu.VMEM((CROWS * LW,), jnp.float32),
            pltpu.VMEM((CROWS * LW, HID), jnp.float32),
            pltpu.VMEM((CROWS * LW, HID), jnp.float32),
            pltpu.VMEM((ZROWS, HID), jnp.float32),
            pltpu.VMEM_SHARED((N_PAD, HID), jnp.float32),
            pltpu.SemaphoreType.DMA,
            pltpu.SemaphoreType.DMA,
            pltpu.SemaphoreType.DMA,
            pltpu.SemaphoreType.DMA,
        ],
    )(y, row, col, ew)


def _dinv(degt):
    deg = jnp.sum(degt, axis=1, keepdims=True) + 1.0
    return jnp.where(deg > 0, lax.rsqrt(deg), 0.0)


def _y1_body(x_ref, w_ref, degt_ref, o_ref):
    dinv = _dinv(degt_ref[...])
    xw = jnp.dot(x_ref[...], w_ref[...],
                 preferred_element_type=jnp.float32)
    o_ref[pl.ds(0, N_NODES)] = dinv[:N_NODES] * xw
    o_ref[pl.ds(N_NODES, N_PAD - N_NODES)] = jnp.zeros(
        (N_PAD - N_NODES, HID), jnp.float32)


def _comb1_body(acc_ref, y_ref, degt_ref, b_ref, w_ref, o_ref):
    dinv = _dinv(degt_ref[...])
    tot = acc_ref[0] + acc_ref[1] + y_ref[...]
    h = jnp.maximum(dinv * tot + b_ref[...], 0.0)
    o_ref[...] = dinv * jnp.dot(h, w_ref[...],
                                preferred_element_type=jnp.float32)


def _comb2_body(acc_ref, y_ref, degt_ref, b_ref, o_ref):
    dinv = _dinv(degt_ref[...])
    tot = acc_ref[0] + acc_ref[1] + y_ref[...]
    o_ref[...] = jnp.maximum(dinv * tot + b_ref[...], 0.0)


def _tc(body, out_shape, *args):
    return pl.pallas_call(
        body, out_shape=jax.ShapeDtypeStruct(out_shape, jnp.float32))(*args)


def kernel(x, edge_index, edge_weight, batch, W1, b1, W2, b2):
    del batch
    x = x.astype(jnp.float32)
    e = edge_weight.shape[0]
    ep = -(-e // (NW * CROWS * LW)) * (NW * CROWS * LW)
    npadex = ep - e
    pad_nodes = (jnp.arange(npadex, dtype=jnp.int32) % (N_PAD - N_NODES)
                 + N_NODES)
    row = jnp.concatenate([edge_index[0], pad_nodes])
    col = jnp.concatenate([edge_index[1], pad_nodes])
    ew = jnp.concatenate(
        [edge_weight.astype(jnp.float32), jnp.zeros((npadex,), jnp.float32)])

    degp = _deg_call(col, ew)
    degt = degp.T
    y1 = _tc(_y1_body, (N_PAD, HID), x, W1, degt)
    acc1 = _edge_call(y1, row, col, ew)
    y2 = _tc(_comb1_body, (N_PAD, HID), acc1, y1, degt,
             b1.reshape(1, HID).astype(jnp.float32), W2)
    acc2 = _edge_call(y2, row, col, ew)
    out = _tc(_comb2_body, (N_PAD, HID), acc2, y2, degt,
              b2.reshape(1, HID).astype(jnp.float32))
    return out[:N_NODES]


# speedup vs baseline: 57.2988x; 1.0204x over previous
import jax
import jax.numpy as jnp
from jax import lax
from jax.experimental import pallas as pl
from jax.experimental.pallas import tpu as pltpu
from jax.experimental.pallas import tpu_sc as plsc

N_NODES = 10000
N_PAD = 10240
HID = 16
NC = 2
NS = 16
NW = NC * NS
CE = 2000
ZROWS = N_PAD // NS

_MESH = plsc.VectorSubcoreMesh(core_axis_name="c", subcore_axis_name="s")
_SC_PARAMS = pltpu.CompilerParams(use_tc_tiling_on_sc=False)


def _deg_call(col, ew):
    rpt = col.shape[0] // NW
    nchunk = rpt // CE

    def body(col_hbm, ew_hbm, out_hbm, colv0, colv1, ewv0, ewv1, zb, degsp,
             dsem0, dsem1):
        colv = [colv0, colv1]
        ewv = [ewv0, ewv1]
        dsem = [dsem0, dsem1]
        c = lax.axis_index("c")
        s = lax.axis_index("s")
        wid = c * NS + s

        @pl.loop(0, ZROWS, step=16)
        def _zero(i):
            zb[pl.ds(i, 16)] = jnp.zeros((16,), jnp.float32)

        pltpu.sync_copy(zb, degsp.at[pl.ds(s * ZROWS, ZROWS)])
        plsc.subcore_barrier()

        base = wid * rpt

        def issue_idx(k):
            p = k & 1
            pltpu.sync_copy(col_hbm.at[pl.ds(base + k * CE, CE)], colv[p])
            pltpu.sync_copy(ew_hbm.at[pl.ds(base + k * CE, CE)], ewv[p])

        issue_idx(0)
        sd = {}
        for k in range(nchunk):
            if k - 1 in sd:
                for d in sd.pop(k - 1):
                    d.wait()
            if k + 1 < nchunk:
                issue_idx(k + 1)
            p = k & 1
            sd[k] = [pltpu.async_copy(ewv[p], degsp.at[colv[p]], dsem[p],
                                      add=True)]
        for kk in sorted(sd):
            for d in sd[kk]:
                d.wait()

        plsc.subcore_barrier()
        pltpu.sync_copy(degsp.at[pl.ds(s * ZROWS, ZROWS)],
                        out_hbm.at[c, pl.ds(s * ZROWS, ZROWS)])

    return pl.kernel(
        body,
        out_type=jax.ShapeDtypeStruct((NC, N_PAD), jnp.float32),
        mesh=_MESH,
        compiler_params=_SC_PARAMS,
        scratch_types=[
            pltpu.VMEM((CE,), jnp.int32),
            pltpu.VMEM((CE,), jnp.int32),
            pltpu.VMEM((CE,), jnp.float32),
            pltpu.VMEM((CE,), jnp.float32),
            pltpu.VMEM((ZROWS,), jnp.float32),
            pltpu.VMEM_SHARED((N_PAD,), jnp.float32),
            pltpu.SemaphoreType.DMA,
            pltpu.SemaphoreType.DMA,
        ],
    )(col, ew)


def _edge_call(y, row, col, ew):
    rpt = col.shape[0] // NW
    nchunk = rpt // CE

    def body(y_hbm, row_hbm, col_hbm, ew_hbm, out_hbm,
             rowv0, rowv1, colv0, colv1, ewv0, ewv1, msg0, msg1,
             zb, accsp, gsem0, gsem1, ssem0, ssem1):
        rowv = [rowv0, rowv1]
        colv = [colv0, colv1]
        ewv = [ewv0, ewv1]
        msg = [msg0, msg1]
        gsem = [gsem0, gsem1]
        ssem = [ssem0, ssem1]
        c = lax.axis_index("c")
        s = lax.axis_index("s")
        wid = c * NS + s

        @pl.loop(0, ZROWS)
        def _zero(i):
            zb[i, :] = jnp.zeros((HID,), jnp.float32)

        pltpu.sync_copy(zb, accsp.at[pl.ds(s * ZROWS, ZROWS)])
        plsc.subcore_barrier()

        base = wid * rpt

        def issue_idx(k):
            p = k & 1
            pltpu.sync_copy(row_hbm.at[pl.ds(base + k * CE, CE)], rowv[p])
            pltpu.sync_copy(col_hbm.at[pl.ds(base + k * CE, CE)], colv[p])
            pltpu.sync_copy(ew_hbm.at[pl.ds(base + k * CE, CE)], ewv[p])

        def issue_gather(k):
            p = k & 1
            return [pltpu.async_copy(y_hbm.at[rowv[p]], msg[p], gsem[p])]

        def issue_scatter(k):
            p = k & 1
            return [pltpu.async_copy(msg[p], accsp.at[colv[p]], ssem[p],
                                     add=True)]

        def scale(k):
            p = k & 1

            @pl.loop(0, CE, step=16)
            def _si(i0):
                ew16 = ewv[p][pl.ds(i0, 16)]
                for t in range(16):
                    msg[p][i0 + t] = msg[p][i0 + t] * ew16[t]

        issue_idx(0)
        gd = {0: issue_gather(0)}
        sd = {}
        for k in range(nchunk):
            if k + 1 < nchunk:
                if k - 1 in sd:
                    for d in sd.pop(k - 1):
                        d.wait()
                issue_idx(k + 1)
                gd[k + 1] = issue_gather(k + 1)
            for d in gd.pop(k):
                d.wait()
            scale(k)
            sd[k] = issue_scatter(k)
        for kk in sorted(sd):
            for d in sd[kk]:
                d.wait()

        plsc.subcore_barrier()
        pltpu.sync_copy(accsp.at[pl.ds(s * ZROWS, ZROWS)],
                        out_hbm.at[c, pl.ds(s * ZROWS, ZROWS)])

    return pl.kernel(
        body,
        out_type=jax.ShapeDtypeStruct((NC, N_PAD, HID), jnp.float32),
        mesh=_MESH,
        compiler_params=_SC_PARAMS,
        scratch_types=[
            pltpu.VMEM((CE,), jnp.int32),
            pltpu.VMEM((CE,), jnp.int32),
            pltpu.VMEM((CE,), jnp.int32),
            pltpu.VMEM((CE,), jnp.int32),
            pltpu.VMEM((CE,), jnp.float32),
            pltpu.VMEM((CE,), jnp.float32),
            pltpu.VMEM((CE, HID), jnp.float32),
            pltpu.VMEM((CE, HID), jnp.float32),
            pltpu.VMEM((ZROWS, HID), jnp.float32),
            pltpu.VMEM_SHARED((N_PAD, HID), jnp.float32),
            pltpu.SemaphoreType.DMA,
            pltpu.SemaphoreType.DMA,
            pltpu.SemaphoreType.DMA,
            pltpu.SemaphoreType.DMA,
        ],
    )(y, row, col, ew)


def _dinv(degt):
    deg = jnp.sum(degt, axis=1, keepdims=True) + 1.0
    return jnp.where(deg > 0, lax.rsqrt(deg), 0.0)[:N_NODES]


def _y1_body(x_ref, w_ref, degt_ref, o_ref):
    xw = jnp.dot(x_ref[...], w_ref[...],
                 preferred_element_type=jnp.float32)
    o_ref[...] = _dinv(degt_ref[...]) * xw


def _comb1_body(acc_ref, y_ref, degt_ref, b_ref, w_ref, o_ref):
    dinv = _dinv(degt_ref[...])
    tot = (acc_ref[0] + acc_ref[1])[:N_NODES] + y_ref[...]
    h = jnp.maximum(dinv * tot + b_ref[...], 0.0)
    o_ref[...] = dinv * jnp.dot(h, w_ref[...],
                                preferred_element_type=jnp.float32)


def _comb2_body(acc_ref, y_ref, degt_ref, b_ref, o_ref):
    dinv = _dinv(degt_ref[...])
    tot = (acc_ref[0] + acc_ref[1])[:N_NODES] + y_ref[...]
    o_ref[...] = jnp.maximum(dinv * tot + b_ref[...], 0.0)


def _tc(body, *args):
    return pl.pallas_call(
        body,
        out_shape=jax.ShapeDtypeStruct((N_NODES, HID), jnp.float32))(*args)


def kernel(x, edge_index, edge_weight, batch, W1, b1, W2, b2):
    del batch
    x = x.astype(jnp.float32)
    e = edge_weight.shape[0]
    ep = -(-e // (NW * CE)) * (NW * CE)
    row = edge_index[0]
    col = edge_index[1]
    ew = edge_weight.astype(jnp.float32)
    if ep != e:
        npadex = ep - e
        spread = jnp.arange(npadex, dtype=jnp.int32) % (N_PAD - N_NODES)
        row = jnp.concatenate([row, spread])
        col = jnp.concatenate([col, spread + N_NODES])
        ew = jnp.concatenate([ew, jnp.zeros((npadex,), jnp.float32)])

    degp = _deg_call(col, ew)
    degt = degp.T
    y1 = _tc(_y1_body, x, W1, degt)
    acc1 = _edge_call(y1, row, col, ew)
    y2 = _tc(_comb1_body, acc1, y1, degt,
             b1.reshape(1, HID).astype(jnp.float32), W2)
    acc2 = _edge_call(y2, row, col, ew)
    return _tc(_comb2_body, acc2, y2, degt,
               b2.reshape(1, HID).astype(jnp.float32))


# speedup vs baseline: 76.3414x; 1.3323x over previous
import jax
import jax.numpy as jnp
from jax import lax
from jax.experimental import pallas as pl
from jax.experimental.pallas import tpu as pltpu
from jax.experimental.pallas import tpu_sc as plsc

N_NODES = 10000
N_PAD = 10240
HID = 16
NC = 2
NS = 16
NW = NC * NS
CE = 2000
ZROWS = N_PAD // NS

_MESH = plsc.VectorSubcoreMesh(core_axis_name="c", subcore_axis_name="s")
_SC_PARAMS = pltpu.CompilerParams(use_tc_tiling_on_sc=False)


def _deg_call(col, ew):
    rpt = col.shape[0] // NW
    nchunk = rpt // CE

    def body(col_hbm, ew_hbm, out_hbm, colv0, colv1, ewv0, ewv1, zb, dbuf,
             dx, degsp, dsem0, dsem1):
        colv = [colv0, colv1]
        ewv = [ewv0, ewv1]
        dsem = [dsem0, dsem1]
        c = lax.axis_index("c")
        s = lax.axis_index("s")
        wid = c * NS + s

        @pl.loop(0, ZROWS, step=16)
        def _zero(i):
            zb[pl.ds(i, 16)] = jnp.zeros((16,), jnp.float32)

        pltpu.sync_copy(zb, degsp.at[pl.ds(s * ZROWS, ZROWS)])
        plsc.subcore_barrier()

        base = wid * rpt

        def issue_idx(k):
            p = k & 1
            pltpu.sync_copy(col_hbm.at[pl.ds(base + k * CE, CE)], colv[p])
            pltpu.sync_copy(ew_hbm.at[pl.ds(base + k * CE, CE)], ewv[p])

        issue_idx(0)
        sd = {}
        for k in range(nchunk):
            if k - 1 in sd:
                for d in sd.pop(k - 1):
                    d.wait()
            if k + 1 < nchunk:
                issue_idx(k + 1)
            p = k & 1
            sd[k] = [pltpu.async_copy(ewv[p], degsp.at[colv[p]], dsem[p],
                                      add=True)]
        for kk in sorted(sd):
            for d in sd[kk]:
                d.wait()

        plsc.subcore_barrier()
        pltpu.sync_copy(degsp.at[pl.ds(s * ZROWS, ZROWS)], dbuf)

        @pl.loop(0, ZROWS, step=16)
        def _exp(g):
            d16 = dbuf[pl.ds(g, 16)]
            for t in range(16):
                dx[g + t] = jnp.ones((HID,), jnp.float32) * d16[t]

        pltpu.sync_copy(dx, out_hbm.at[c, pl.ds(s * ZROWS, ZROWS)])

    return pl.kernel(
        body,
        out_type=jax.ShapeDtypeStruct((NC, N_PAD, HID), jnp.float32),
        mesh=_MESH,
        compiler_params=_SC_PARAMS,
        scratch_types=[
            pltpu.VMEM((CE,), jnp.int32),
            pltpu.VMEM((CE,), jnp.int32),
            pltpu.VMEM((CE,), jnp.float32),
            pltpu.VMEM((CE,), jnp.float32),
            pltpu.VMEM((ZROWS,), jnp.float32),
            pltpu.VMEM((ZROWS,), jnp.float32),
            pltpu.VMEM((ZROWS, HID), jnp.float32),
            pltpu.VMEM_SHARED((N_PAD,), jnp.float32),
            pltpu.SemaphoreType.DMA,
            pltpu.SemaphoreType.DMA,
        ],
    )(col, ew)


def _edge_call(y, row, col, ew):
    rpt = col.shape[0] // NW
    nchunk = rpt // CE

    def body(y_hbm, row_hbm, col_hbm, ew_hbm, out_hbm,
             rowv0, rowv1, colv0, colv1, ewv0, ewv1, msg0, msg1,
             zb, accsp, gsem0, gsem1, ssem0, ssem1):
        rowv = [rowv0, rowv1]
        colv = [colv0, colv1]
        ewv = [ewv0, ewv1]
        msg = [msg0, msg1]
        gsem = [gsem0, gsem1]
        ssem = [ssem0, ssem1]
        c = lax.axis_index("c")
        s = lax.axis_index("s")
        wid = c * NS + s

        @pl.loop(0, ZROWS)
        def _zero(i):
            zb[i, :] = jnp.zeros((HID,), jnp.float32)

        pltpu.sync_copy(zb, accsp.at[pl.ds(s * ZROWS, ZROWS)])
        plsc.subcore_barrier()

        base = wid * rpt

        def issue_idx(k):
            p = k & 1
            pltpu.sync_copy(row_hbm.at[pl.ds(base + k * CE, CE)], rowv[p])
            pltpu.sync_copy(col_hbm.at[pl.ds(base + k * CE, CE)], colv[p])
            pltpu.sync_copy(ew_hbm.at[pl.ds(base + k * CE, CE)], ewv[p])

        def issue_gather(k):
            p = k & 1
            return [pltpu.async_copy(y_hbm.at[rowv[p]], msg[p], gsem[p])]

        def issue_scatter(k):
            p = k & 1
            return [pltpu.async_copy(msg[p], accsp.at[colv[p]], ssem[p],
                                     add=True)]

        def scale(k):
            p = k & 1

            @pl.loop(0, CE, step=16)
            def _si(i0):
                ew16 = ewv[p][pl.ds(i0, 16)]
                for t in range(16):
                    msg[p][i0 + t] = msg[p][i0 + t] * ew16[t]

        issue_idx(0)
        gd = {0: issue_gather(0)}
        sd = {}
        for k in range(nchunk):
            if k + 1 < nchunk:
                if k - 1 in sd:
                    for d in sd.pop(k - 1):
                        d.wait()
                issue_idx(k + 1)
                gd[k + 1] = issue_gather(k + 1)
            for d in gd.pop(k):
                d.wait()
            scale(k)
            sd[k] = issue_scatter(k)
        for kk in sorted(sd):
            for d in sd[kk]:
                d.wait()

        plsc.subcore_barrier()
        pltpu.sync_copy(accsp.at[pl.ds(s * ZROWS, ZROWS)],
                        out_hbm.at[c, pl.ds(s * ZROWS, ZROWS)])

    return pl.kernel(
        body,
        out_type=jax.ShapeDtypeStruct((NC, N_PAD, HID), jnp.float32),
        mesh=_MESH,
        compiler_params=_SC_PARAMS,
        scratch_types=[
            pltpu.VMEM((CE,), jnp.int32),
            pltpu.VMEM((CE,), jnp.int32),
            pltpu.VMEM((CE,), jnp.int32),
            pltpu.VMEM((CE,), jnp.int32),
            pltpu.VMEM((CE,), jnp.float32),
            pltpu.VMEM((CE,), jnp.float32),
            pltpu.VMEM((CE, HID), jnp.float32),
            pltpu.VMEM((CE, HID), jnp.float32),
            pltpu.VMEM((ZROWS, HID), jnp.float32),
            pltpu.VMEM_SHARED((N_PAD, HID), jnp.float32),
            pltpu.SemaphoreType.DMA,
            pltpu.SemaphoreType.DMA,
            pltpu.SemaphoreType.DMA,
            pltpu.SemaphoreType.DMA,
        ],
    )(y, row, col, ew)


NPK = N_NODES // 8


def _dinvx(degx_ref):
    deg = (degx_ref[0] + degx_ref[1])[:NPK] + 1.0
    return jnp.where(deg > 0, lax.rsqrt(deg), 0.0)


def _y1_body(x_ref, w_ref, degx_ref, o_ref):
    xw = jnp.dot(x_ref[...], w_ref[...],
                 preferred_element_type=jnp.float32)
    o_ref[...] = _dinvx(degx_ref) * xw


def _comb1_body(acc_ref, y_ref, degx_ref, b_ref, w_ref, o_ref):
    dinv = _dinvx(degx_ref)
    tot = (acc_ref[0] + acc_ref[1])[:NPK] + y_ref[...]
    h = jnp.maximum(dinv * tot + b_ref[...], 0.0)
    o_ref[...] = dinv * jnp.dot(h, w_ref[...],
                                preferred_element_type=jnp.float32)


def _comb2_body(acc_ref, y_ref, degx_ref, b_ref, o_ref):
    dinv = _dinvx(degx_ref)
    tot = (acc_ref[0] + acc_ref[1])[:NPK] + y_ref[...]
    o_ref[...] = jnp.maximum(dinv * tot + b_ref[...], 0.0)


def _tc(body, *args):
    return pl.pallas_call(
        body,
        out_shape=jax.ShapeDtypeStruct((NPK, 128), jnp.float32))(*args)


def kernel(x, edge_index, edge_weight, batch, W1, b1, W2, b2):
    del batch
    x = x.astype(jnp.float32)
    e = edge_weight.shape[0]
    ep = -(-e // (NW * CE)) * (NW * CE)
    row = edge_index[0]
    col = edge_index[1]
    ew = edge_weight.astype(jnp.float32)
    if ep != e:
        npadex = ep - e
        spread = jnp.arange(npadex, dtype=jnp.int32) % (N_PAD - N_NODES)
        row = jnp.concatenate([row, spread])
        col = jnp.concatenate([col, spread + N_NODES])
        ew = jnp.concatenate([ew, jnp.zeros((npadex,), jnp.float32)])

    eye8 = jnp.eye(8, dtype=jnp.float32)
    w1blk = jnp.kron(eye8, W1.astype(jnp.float32))
    w2blk = jnp.kron(eye8, W2.astype(jnp.float32))
    b1x = jnp.tile(b1.astype(jnp.float32), 8).reshape(1, 128)
    b2x = jnp.tile(b2.astype(jnp.float32), 8).reshape(1, 128)
    xp = x.reshape(N_NODES // 8, 8 * x.shape[1])

    degx = _deg_call(col, ew).reshape(NC, N_PAD // 8, 128)
    y1p = _tc(_y1_body, xp, w1blk, degx)
    acc1 = _edge_call(y1p.reshape(N_NODES, HID), row, col, ew)
    y2p = _tc(_comb1_body, acc1.reshape(NC, N_PAD // 8, 128), y1p, degx,
              b1x, w2blk)
    acc2 = _edge_call(y2p.reshape(N_NODES, HID), row, col, ew)
    outp = _tc(_comb2_body, acc2.reshape(NC, N_PAD // 8, 128), y2p, degx,
               b2x)
    return outp.reshape(N_NODES, HID)
